# Initial kernel scaffold; baseline (speedup 1.0000x reference)
import jax
import jax.numpy as jnp
from jax.experimental import pallas as pl


def kernel(x_0, x_1, x_2, adj_0_0, adj_0_1, adj_1_0, adj_1_1, adj_1_2, adj_2_1, inv_0_0, inv_0_1, inv_1_0, inv_1_1, inv_1_2, inv_2_1, pos_0, vel_0, params):
    raise NotImplementedError("write your pallas kernel here")


# speedup vs baseline: 1.4952x; 1.4952x over previous
import functools

import jax
import jax.numpy as jnp
from jax import lax
from jax.experimental import pallas as pl
from jax.experimental.pallas import tpu as pltpu
from jax.experimental.pallas import tpu_sc as plsc

f32 = jnp.float32
i32 = jnp.int32

H = 128
_N = {'0': 10000, '1': 30000, '2': 10000}
_ADJS = [
    ('0_0', '0', '0', 160000, 3, 128),
    ('0_1', '0', '1', 60000, 3, 120),
    ('1_0', '1', '0', 60000, 3, 120),
    ('1_1', '1', '1', 120000, 6, 120),
    ('1_2', '1', '2', 30000, 6, 120),
    ('2_1', '2', '1', 30000, 6, 120),
]
_NW = 32
_SPR = 30080
_BT = 1000


def _silu(v):
    return v * jax.nn.sigmoid(v)


def _mesh():
    return plsc.VectorSubcoreMesh(core_axis_name='c', subcore_axis_name='s')


_SC_PARAMS = pltpu.CompilerParams(needs_layout_passes=False,
                                  use_tc_tiling_on_sc=False)


def _t1_body(nout, x_ref, w_ref, *outs):
    acc = jnp.dot(x_ref[:], w_ref[:], preferred_element_type=f32)
    for k in range(nout):
        outs[k][:] = acc[:, k * H:(k + 1) * H]


def _t1(x, wcat, nout):
    n = x.shape[0]
    return pl.pallas_call(
        functools.partial(_t1_body, nout),
        grid=(n // _BT,),
        in_specs=[pl.BlockSpec((_BT, H), lambda i: (i, 0)),
                  pl.BlockSpec(wcat.shape, lambda i: (0, 0))],
        out_specs=[pl.BlockSpec((_BT, H), lambda i: (i, 0))] * nout,
        out_shape=[jax.ShapeDtypeStruct((n, H), f32)] * nout,
    )(x, wcat)


def _s1_body(*refs):
    ins = refs[:24]
    outs = refs[24:36]
    iv0, iv1, ra, rb, sa, sb = refs[36:]
    wid = lax.axis_index('c') * 16 + lax.axis_index('s')
    z16 = jnp.zeros((16,), i32)
    for k, (a, si, sj, ne, ninv, c) in enumerate(_ADJS):
        tbl_i, tbl_j, idx0, idx1 = ins[4 * k:4 * k + 4]
        oi, oj = outs[2 * k:2 * k + 2]
        if c < 128:
            iv0[pl.ds(112, 16)] = z16
            iv1[pl.ds(112, 16)] = z16
        nch = ne // c

        def body(t, _, idx0=idx0, idx1=idx1, tbl_i=tbl_i, tbl_j=tbl_j,
                 oi=oi, oj=oj, c=c):
            base = pl.multiple_of((wid + t * _NW) * c, 8)
            pltpu.sync_copy(idx0.at[pl.ds(base, c)], iv0.at[pl.ds(0, c)])
            pltpu.sync_copy(idx1.at[pl.ds(base, c)], iv1.at[pl.ds(0, c)])
            ga = pltpu.async_copy(tbl_i.at[iv0], ra, sa)
            gb = pltpu.async_copy(tbl_j.at[iv1], rb, sb)
            ga.wait()
            gb.wait()
            pltpu.sync_copy(ra.at[pl.ds(0, c)], oi.at[pl.ds(base, c)])
            pltpu.sync_copy(rb.at[pl.ds(0, c)], oj.at[pl.ds(base, c)])
            return 0

        lax.fori_loop(0, (nch - wid + _NW - 1) // _NW, body, 0)


def _s1_call(args, out_type):
    fn = pl.kernel(
        _s1_body, out_type=out_type, mesh=_mesh(),
        compiler_params=_SC_PARAMS,
        scratch_types=[
            pltpu.VMEM((128,), i32), pltpu.VMEM((128,), i32),
            pltpu.VMEM((128, H), f32), pltpu.VMEM((128, H), f32),
            pltpu.SemaphoreType.DMA, pltpu.SemaphoreType.DMA,
        ])
    return fn(*args)


def _t2_body(has_pm, *refs):
    xi, xj, inv, w1c, b1, w2, b2, eiw, eib = refs[:9]
    s = xi[:] + xj[:] + jnp.dot(inv[:], w1c[:], preferred_element_type=f32) + b1[:]
    h1 = _silu(s)
    m = _silu(jnp.dot(h1, w2[:], preferred_element_type=f32) + b2[:])
    ew = jax.nn.sigmoid(jnp.dot(m, eiw[:], preferred_element_type=f32) + eib[0, 0])
    if has_pm:
        p1w, p1b, p2w, p2b, out, pm = refs[9:]
        t = _silu(jnp.dot(m, p1w[:], preferred_element_type=f32) + p1b[:])
        pm[:] = jnp.dot(t, p2w[:], preferred_element_type=f32) + p2b[0, 0]
    else:
        out, = refs[9:]
    out[:] = m * ew


def _t2(a, ne, ninv, args):
    has_pm = a == '0_0'
    wspec = lambda arr: pl.BlockSpec(arr.shape, lambda i: tuple(0 for _ in arr.shape))
    in_specs = [pl.BlockSpec((_BT, H), lambda i: (i, 0)),
                pl.BlockSpec((_BT, H), lambda i: (i, 0)),
                pl.BlockSpec((_BT, ninv), lambda i: (i, 0))]
    in_specs += [wspec(w) for w in args[3:]]
    out_specs = [pl.BlockSpec((_BT, H), lambda i: (i, 0))]
    out_shape = [jax.ShapeDtypeStruct((ne, H), f32)]
    if has_pm:
        out_specs.append(pl.BlockSpec((_BT, 1), lambda i: (i, 0)))
        out_shape.append(jax.ShapeDtypeStruct((ne, 1), f32))
    return pl.pallas_call(
        functools.partial(_t2_body, has_pm),
        grid=(ne // _BT,),
        in_specs=in_specs, out_specs=out_specs, out_shape=out_shape,
    )(*args)


def _s2_body(*refs):
    ins = refs[:12]
    outs = refs[12:18]
    rows, idxv, spm = refs[18:]
    cid = lax.axis_index('c')
    sid = lax.axis_index('s')
    lane0 = cid * 64
    zf = jnp.zeros((16,), f32)

    for k, (a, si, sj, ne, ninv, c) in enumerate(_ADJS):
        contrib, idx1 = ins[2 * k], ins[2 * k + 1]
        agg = outs[k]
        ndst = _N[sj]
        nch = ne // c

        def zr(r, _):
            for g in range(4):
                rows[r, pl.ds(g * 16, 16)] = zf
            return 0

        lax.fori_loop(0, 128, zr, 0)
        bz = sid * 1880
        for j in range(14):
            pltpu.sync_copy(rows, spm.at[pl.ds(bz + j * 128, 128)])
        pltpu.sync_copy(rows.at[pl.ds(0, 88)], spm.at[pl.ds(bz + 1792, 88)])
        if c < 128:
            idxv[pl.ds(112, 16)] = jnp.full((16,), ndst, i32)
        plsc.subcore_barrier()

        def body(t, _, contrib=contrib, idx1=idx1, c=c):
            off = pl.multiple_of((sid + t * 16) * c, 8)
            pltpu.sync_copy(idx1.at[pl.ds(off, c)], idxv.at[pl.ds(0, c)])
            pltpu.sync_copy(contrib.at[pl.ds(off, c), pl.ds(lane0, 64)],
                            rows.at[pl.ds(0, c)])
            pltpu.sync_copy(rows, spm.at[idxv], add=True)
            return 0

        lax.fori_loop(0, (nch - sid + 15) // 16, body, 0)
        plsc.subcore_barrier()
        nchd = ndst // 40

        def dump(t, _, agg=agg):
            r0 = pl.multiple_of((sid + t * 16) * 40, 8)
            pltpu.sync_copy(spm.at[pl.ds(r0, 40)],
                            agg.at[pl.ds(r0, 40), pl.ds(lane0, 64)])
            return 0

        lax.fori_loop(0, (nchd - sid + 15) // 16, dump, 0)
        plsc.subcore_barrier()


def _s2_call(args, out_type):
    fn = pl.kernel(
        _s2_body, out_type=out_type, mesh=_mesh(),
        compiler_params=_SC_PARAMS,
        scratch_types=[
            pltpu.VMEM((128, 64), f32), pltpu.VMEM((128,), i32),
            pltpu.VMEM_SHARED((_SPR, 64), f32),
        ])
    return fn(*args)


def _s3_body(pm, idx0, idx1, px, py, pz, out,
             pxv, pyv, pzv, acc, ivb0, ivb1, pmv, ridx, ash):
    cid = lax.axis_index('c')
    sid = lax.axis_index('s')

    @pl.when(cid == 0)
    def _():
        zf = jnp.zeros((16,), f32)
        pltpu.sync_copy(px, pxv)
        pltpu.sync_copy(py, pyv)
        pltpu.sync_copy(pz, pzv)

        def za(r, _):
            for g in range(8):
                acc[r, pl.ds(g * 16, 16)] = zf
            return 0

        lax.fori_loop(0, 256, za, 0)
        pltpu.sync_copy(acc.at[pl.ds(0, 16)], ash.at[pl.ds(sid * 16, 16)])
        plsc.subcore_barrier()

        def body(t, _):
            off = pl.multiple_of((sid + t * 16) * 128, 8)
            pltpu.sync_copy(idx0.at[pl.ds(off, 128)], ivb0)
            pltpu.sync_copy(idx1.at[pl.ds(off, 128)], ivb1)
            pltpu.sync_copy(pm.at[pl.ds(off, 128)], pmv)
            for g in range(8):
                s_ = ivb0[pl.ds(g * 16, 16)]
                r_ = ivb1[pl.ds(g * 16, 16)]
                w_ = pmv[pl.ds(g * 16, 16)]
                srow = lax.shift_right_logical(s_, 7)
                scol = s_ & 127
                row = lax.shift_right_logical(r_, 7)
                col = r_ & 127
                dx = (plsc.load_gather(pxv, [srow, scol])
                      - plsc.load_gather(pxv, [row, col])) * w_
                dy = (plsc.load_gather(pyv, [srow, scol])
                      - plsc.load_gather(pyv, [row, col])) * w_
                dz = (plsc.load_gather(pzv, [srow, scol])
                      - plsc.load_gather(pzv, [row, col])) * w_
                plsc.addupdate_scatter(acc, [row, col], dx)
                plsc.addupdate_scatter(acc, [row + 80, col], dy)
                plsc.addupdate_scatter(acc, [row + 160, col], dz)
            return 0

        lax.fori_loop(0, (1250 - sid + 15) // 16, body, 0)
        for t in range(2):
            for g in range(8):
                ridx[pl.ds(g * 16, 16)] = lax.iota(i32, 16) + (t * 128 + g * 16)
            pltpu.sync_copy(acc.at[pl.ds(t * 128, 128)], ash.at[ridx], add=True)
        plsc.subcore_barrier()

        @pl.when(sid == 0)
        def _dump():
            pltpu.sync_copy(ash, out)


def _s3_call(pm, idx0, idx1, px, py, pz):
    fn = pl.kernel(
        _s3_body, out_type=jax.ShapeDtypeStruct((256, H), f32), mesh=_mesh(),
        compiler_params=_SC_PARAMS,
        scratch_types=[
            pltpu.VMEM((80, H), f32), pltpu.VMEM((80, H), f32),
            pltpu.VMEM((80, H), f32), pltpu.VMEM((256, H), f32),
            pltpu.VMEM((128,), i32), pltpu.VMEM((128,), i32),
            pltpu.VMEM((128,), f32), pltpu.VMEM((128,), i32),
            pltpu.VMEM_SHARED((256, H), f32),
        ])
    return fn(pm, idx0, idx1, px, py, pz)


def _t3_body(nagg, has_vel, *refs):
    x = refs[0]
    aggs = refs[1:1 + nagg]
    u1w, u1b, u2w, u2b, g, b = refs[1 + nagg:7 + nagg]
    rest = refs[7 + nagg:]
    cat = jnp.concatenate([x[:]] + [a[:] for a in aggs], axis=1)
    h = _silu(jnp.dot(cat, u1w[:], preferred_element_type=f32) + u1b[:])
    h = jnp.dot(h, u2w[:], preferred_element_type=f32) + u2b[:]
    v = x[:] + h
    mu = jnp.mean(v, axis=1, keepdims=True)
    var = jnp.mean((v - mu) * (v - mu), axis=1, keepdims=True)
    xn = (v - mu) * lax.rsqrt(var + 1e-5) * g[:] + b[:]
    if has_vel:
        v1w, v1b, v2w, v2b, pos8, s8, vel8, xn_ref, loc_ref = rest
        xn_ref[:] = xn
        vm = jnp.dot(_silu(jnp.dot(xn, v1w[:], preferred_element_type=f32) + v1b[:]),
                     v2w[:], preferred_element_type=f32) + v2b[0, 0]
        loc_ref[:] = pos8[:] + s8[:] + vm * vel8[:]
    else:
        xn_ref, = rest
        xn_ref[:] = xn


def _t3(d, x, aggs, p, extra=None):
    n = _N[d]
    nagg = len(aggs)
    has_vel = extra is not None
    args = [x] + aggs + [
        p['up1_' + d + '_w'], p['up1_' + d + '_b'].reshape(1, H),
        p['up2_' + d + '_w'], p['up2_' + d + '_b'].reshape(1, H),
        p['ln_g_' + d].reshape(1, H), p['ln_b_' + d].reshape(1, H)]
    wspec = lambda arr: pl.BlockSpec(arr.shape, lambda i: tuple(0 for _ in arr.shape))
    in_specs = [pl.BlockSpec((_BT, H), lambda i: (i, 0))] * (1 + nagg)
    in_specs += [wspec(w) for w in args[1 + nagg:]]
    out_specs = [pl.BlockSpec((_BT, H), lambda i: (i, 0))]
    out_shape = [jax.ShapeDtypeStruct((n, H), f32)]
    if has_vel:
        pos8, s8, vel8 = extra
        args += [p['vel1_w'], p['vel1_b'].reshape(1, H),
                 p['vel2_w'], p['vel2_b'].reshape(1, 1), pos8, s8, vel8]
        in_specs += [wspec(p['vel1_w']), wspec(p['vel1_b'].reshape(1, H)),
                     wspec(p['vel2_w']), wspec(p['vel2_b'].reshape(1, 1)),
                     pl.BlockSpec((_BT, 8), lambda i: (i, 0)),
                     pl.BlockSpec((_BT, 8), lambda i: (i, 0)),
                     pl.BlockSpec((_BT, 8), lambda i: (i, 0))]
        out_specs.append(pl.BlockSpec((_BT, 8), lambda i: (i, 0)))
        out_shape.append(jax.ShapeDtypeStruct((n, 8), f32))
    return pl.pallas_call(
        functools.partial(_t3_body, nagg, has_vel),
        grid=(n // _BT,),
        in_specs=in_specs, out_specs=out_specs, out_shape=out_shape,
    )(*args)


def kernel(x_0, x_1, x_2, adj_0_0, adj_0_1, adj_1_0, adj_1_1, adj_1_2, adj_2_1,
           inv_0_0, inv_0_1, inv_1_0, inv_1_1, inv_1_2, inv_2_1,
           pos_0, vel_0, params):
    p = params
    x = {'0': x_0, '1': x_1, '2': x_2}
    adj = {'0_0': adj_0_0, '0_1': adj_0_1, '1_0': adj_1_0,
           '1_1': adj_1_1, '1_2': adj_1_2, '2_1': adj_2_1}
    inv = {'0_0': inv_0_0, '0_1': inv_0_1, '1_0': inv_1_0,
           '1_1': inv_1_1, '1_2': inv_1_2, '2_1': inv_2_1}

    groups = {'0': [], '1': [], '2': []}
    for a, si, sj, *_ in _ADJS:
        groups[si].append((a, 'i'))
        groups[sj].append((a, 'j'))
    tbl = {}
    for d in ['0', '1', '2']:
        ws = []
        for a, role in groups[d]:
            w1 = p['mp1_' + a + '_w']
            ws.append(w1[:H] if role == 'i' else w1[H:2 * H])
        outs = _t1(x[d], jnp.concatenate(ws, axis=1), len(ws))
        for (a, role), o in zip(groups[d], outs):
            tbl[(a, role)] = o

    s1_in = []
    s1_out = []
    for a, si, sj, ne, *_ in _ADJS:
        s1_in += [tbl[(a, 'i')], tbl[(a, 'j')], adj[a][0], adj[a][1]]
        s1_out += [jax.ShapeDtypeStruct((ne, H), f32)] * 2
    gath = _s1_call(s1_in, s1_out)

    contrib = {}
    pm = None
    for k, (a, si, sj, ne, ninv, c) in enumerate(_ADJS):
        args = [gath[2 * k], gath[2 * k + 1], inv[a],
                p['mp1_' + a + '_w'][2 * H:], p['mp1_' + a + '_b'].reshape(1, H),
                p['mp2_' + a + '_w'], p['mp2_' + a + '_b'].reshape(1, H),
                p['ei_' + a + '_w'], p['ei_' + a + '_b'].reshape(1, 1)]
        if a == '0_0':
            args += [p['pos1_w'], p['pos1_b'].reshape(1, H),
                     p['pos2_w'], p['pos2_b'].reshape(1, 1)]
            contrib[a], pm = _t2(a, ne, ninv, args)
            pm = pm.reshape(-1)
        else:
            contrib[a], = _t2(a, ne, ninv, args)

    s2_in = []
    s2_out = []
    for a, si, sj, ne, ninv, c in _ADJS:
        s2_in += [contrib[a], adj[a][1]]
        s2_out.append(jax.ShapeDtypeStruct((_N[sj], H), f32))
    aggs = _s2_call(s2_in, s2_out)
    agg = {a: aggs[k] for k, (a, *_r) in enumerate(_ADJS)}

    ppad = jnp.zeros((240,), f32)
    planes = [jnp.concatenate([pos_0[:, k], ppad]).reshape(80, H)
              for k in range(3)]
    sacc = _s3_call(pm, adj['0_0'][0], adj['0_0'][1], *planes)
    flat = sacc.reshape(-1)
    s3 = jnp.stack([flat[0:10000], flat[10240:20240], flat[20480:30480]], axis=1)

    zpad = jnp.zeros((_N['0'], 5), f32)
    pos8 = jnp.concatenate([pos_0, zpad], axis=1)
    s8 = jnp.concatenate([s3, zpad], axis=1)
    vel8 = jnp.concatenate([vel_0, zpad], axis=1)
    xn0, loc8 = _t3('0', x_0, [agg['0_0'], agg['1_0']], p, (pos8, s8, vel8))
    xn1, = _t3('1', x_1, [agg['0_1'], agg['1_1'], agg['2_1']], p)
    xn2, = _t3('2', x_2, [agg['1_2']], p)
    return xn0, xn1, xn2, loc8[:, :3]


# speedup vs baseline: 2.3898x; 1.5983x over previous
import functools

import jax
import jax.numpy as jnp
from jax import lax
from jax.experimental import pallas as pl
from jax.experimental.pallas import tpu as pltpu
from jax.experimental.pallas import tpu_sc as plsc

f32 = jnp.float32
i32 = jnp.int32

H = 128
_N = {'0': 10000, '1': 30000, '2': 10000}
_ADJS = [
    ('0_0', '0', '0', 160000, 3, 128),
    ('0_1', '0', '1', 60000, 3, 120),
    ('1_0', '1', '0', 60000, 3, 120),
    ('1_1', '1', '1', 120000, 6, 120),
    ('1_2', '1', '2', 30000, 6, 120),
    ('2_1', '2', '1', 30000, 6, 120),
]
_NW = 32
_SPR = 30080
_BT = 1000


def _silu(v):
    return v * jax.nn.sigmoid(v)


def _mesh():
    return plsc.VectorSubcoreMesh(core_axis_name='c', subcore_axis_name='s')


_SC_PARAMS = pltpu.CompilerParams(needs_layout_passes=False,
                                  use_tc_tiling_on_sc=False)


def _t1_body(nout, x_ref, w_ref, *outs):
    acc = jnp.dot(x_ref[:], w_ref[:], preferred_element_type=f32)
    for k in range(nout):
        outs[k][:] = acc[:, k * H:(k + 1) * H]


def _t1(x, wcat, nout):
    n = x.shape[0]
    return pl.pallas_call(
        functools.partial(_t1_body, nout),
        grid=(n // _BT,),
        in_specs=[pl.BlockSpec((_BT, H), lambda i: (i, 0)),
                  pl.BlockSpec(wcat.shape, lambda i: (0, 0))],
        out_specs=[pl.BlockSpec((_BT, H), lambda i: (i, 0))] * nout,
        out_shape=[jax.ShapeDtypeStruct((n, H), f32)] * nout,
    )(x, wcat)


def _strip_load(idx2d, strip, s0, nw0, r, wid):
    @pl.when(wid < r)
    def _():
        pltpu.sync_copy(idx2d.at[pl.ds(s0, nw0 + 1)], strip.at[pl.ds(0, nw0 + 1)])

    @pl.when(wid >= r)
    def _():
        pltpu.sync_copy(idx2d.at[pl.ds(s0, nw0)], strip.at[pl.ds(0, nw0)])


def _s1_body(*refs):
    ins = refs[:24]
    outs = refs[24:36]
    (st128a, st128b, st120a, st120b, ra0, ra1, ra2, rb0, rb1, rb2,
     ga0, ga1, ga2, sa0, sa1, sa2) = refs[36:]
    ras = [ra0, ra1, ra2]
    rbs = [rb0, rb1, rb2]
    gas = [ga0, ga1, ga2]
    sas = [sa0, sa1, sa2]
    wid = lax.axis_index('c') * 16 + lax.axis_index('s')
    for k, (a, si, sj, ne, ninv, c) in enumerate(_ADJS):
        tbl_i, tbl_j, idx0, idx1 = ins[4 * k:4 * k + 4]
        oi, oj = outs[2 * k:2 * k + 2]
        s0a, s1a = (st128a, st128b) if c == 128 else (st120a, st120b)
        nch = ne // c
        nw0 = nch // _NW
        r = nch % _NW
        s0 = wid * nw0 + jnp.minimum(wid, r)
        mynw = nw0 + jnp.where(wid < r, 1, 0)
        _strip_load(idx0, s0a, s0, nw0, r, wid)
        _strip_load(idx1, s1a, s0, nw0, r, wid)
        ngr = mynw // 3
        rem = mynw - ngr * 3

        def grp(g, _, tbl_i=tbl_i, tbl_j=tbl_j, oi=oi, oj=oj, c=c,
                s0a=s0a, s1a=s1a, s0=s0):
            for j in range(3):
                i = g * 3 + j

                @pl.when(g > 0)
                def _(j=j, c=c, oi=oi, oj=oj):
                    pltpu.make_async_copy(
                        ras[j].at[pl.ds(0, c)], oi.at[pl.ds(0, c)], sas[j]).wait()
                    pltpu.make_async_copy(
                        rbs[j].at[pl.ds(0, c)], oj.at[pl.ds(0, c)], sas[j]).wait()

                pltpu.async_copy(tbl_i.at[s0a.at[i]], ras[j].at[pl.ds(0, c)], gas[j])
                pltpu.async_copy(tbl_j.at[s1a.at[i]], rbs[j].at[pl.ds(0, c)], gas[j])
            for j in range(3):
                i = g * 3 + j
                pltpu.make_async_copy(
                    tbl_i.at[s0a.at[i]], ras[j].at[pl.ds(0, c)], gas[j]).wait()
                pltpu.make_async_copy(
                    tbl_j.at[s1a.at[i]], rbs[j].at[pl.ds(0, c)], gas[j]).wait()
                off = pl.multiple_of((s0 + i) * c, 8)
                pltpu.async_copy(ras[j].at[pl.ds(0, c)], oi.at[pl.ds(off, c)], sas[j])
                pltpu.async_copy(rbs[j].at[pl.ds(0, c)], oj.at[pl.ds(off, c)], sas[j])
            return 0

        lax.fori_loop(0, ngr, grp, 0)
        for j in range(3):
            @pl.when(ngr > 0)
            def _(j=j, c=c, oi=oi, oj=oj):
                pltpu.make_async_copy(
                    ras[j].at[pl.ds(0, c)], oi.at[pl.ds(0, c)], sas[j]).wait()
                pltpu.make_async_copy(
                    rbs[j].at[pl.ds(0, c)], oj.at[pl.ds(0, c)], sas[j]).wait()

        def remb(t, _, tbl_i=tbl_i, tbl_j=tbl_j, oi=oi, oj=oj, c=c,
                 s0a=s0a, s1a=s1a, s0=s0, ngr=ngr):
            i = ngr * 3 + t
            pltpu.async_copy(tbl_i.at[s0a.at[i]], ras[0].at[pl.ds(0, c)], gas[0])
            pltpu.async_copy(tbl_j.at[s1a.at[i]], rbs[0].at[pl.ds(0, c)], gas[0])
            pltpu.make_async_copy(
                tbl_i.at[s0a.at[i]], ras[0].at[pl.ds(0, c)], gas[0]).wait()
            pltpu.make_async_copy(
                tbl_j.at[s1a.at[i]], rbs[0].at[pl.ds(0, c)], gas[0]).wait()
            off = pl.multiple_of((s0 + i) * c, 8)
            pltpu.sync_copy(ras[0].at[pl.ds(0, c)], oi.at[pl.ds(off, c)])
            pltpu.sync_copy(rbs[0].at[pl.ds(0, c)], oj.at[pl.ds(off, c)])
            return 0

        lax.fori_loop(0, rem, remb, 0)


def _s1_call(args, out_type):
    fn = pl.kernel(
        _s1_body, out_type=out_type, mesh=_mesh(),
        compiler_params=_SC_PARAMS,
        scratch_types=[
            pltpu.VMEM((40, 128), i32), pltpu.VMEM((40, 128), i32),
            pltpu.VMEM((32, 120), i32), pltpu.VMEM((32, 120), i32),
            pltpu.VMEM((128, H), f32), pltpu.VMEM((128, H), f32),
            pltpu.VMEM((128, H), f32), pltpu.VMEM((128, H), f32),
            pltpu.VMEM((128, H), f32), pltpu.VMEM((128, H), f32),
            pltpu.SemaphoreType.DMA, pltpu.SemaphoreType.DMA,
            pltpu.SemaphoreType.DMA, pltpu.SemaphoreType.DMA,
            pltpu.SemaphoreType.DMA, pltpu.SemaphoreType.DMA,
        ])
    return fn(*args)


def _t2_body(has_pm, *refs):
    xi, xj, inv, w1c, b1, w2, b2, eiw, eib = refs[:9]
    s = xi[:] + xj[:] + jnp.dot(inv[:], w1c[:], preferred_element_type=f32) + b1[:]
    h1 = _silu(s)
    m = _silu(jnp.dot(h1, w2[:], preferred_element_type=f32) + b2[:])
    ew = jax.nn.sigmoid(jnp.dot(m, eiw[:], preferred_element_type=f32) + eib[0, 0])
    if has_pm:
        p1w, p1b, p2w, p2b, out, pm = refs[9:]
        t = _silu(jnp.dot(m, p1w[:], preferred_element_type=f32) + p1b[:])
        pm[:] = jnp.dot(t, p2w[:], preferred_element_type=f32) + p2b[0, 0]
    else:
        out, = refs[9:]
    out[:] = m * ew


def _t2(a, ne, ninv, args):
    has_pm = a == '0_0'
    wspec = lambda arr: pl.BlockSpec(arr.shape, lambda i: tuple(0 for _ in arr.shape))
    in_specs = [pl.BlockSpec((_BT, H), lambda i: (i, 0)),
                pl.BlockSpec((_BT, H), lambda i: (i, 0)),
                pl.BlockSpec((_BT, ninv), lambda i: (i, 0))]
    in_specs += [wspec(w) for w in args[3:]]
    out_specs = [pl.BlockSpec((_BT, H), lambda i: (i, 0))]
    out_shape = [jax.ShapeDtypeStruct((ne, H), f32)]
    if has_pm:
        out_specs.append(pl.BlockSpec((_BT, 1), lambda i: (i, 0)))
        out_shape.append(jax.ShapeDtypeStruct((ne, 1), f32))
    return pl.pallas_call(
        functools.partial(_t2_body, has_pm),
        grid=(ne // _BT,),
        in_specs=in_specs, out_specs=out_specs, out_shape=out_shape,
    )(*args)


def _s2_body(*refs):
    ins = refs[:12]
    outs = refs[12:18]
    (st128, st120, r0b, r1b, r2b, spm,
     cp0, cp1, cp2, sc0, sc1, sc2) = refs[18:]
    rows = [r0b, r1b, r2b]
    cps = [cp0, cp1, cp2]
    scs = [sc0, sc1, sc2]
    cid = lax.axis_index('c')
    sid = lax.axis_index('s')
    zf = jnp.zeros((16,), f32)

    for k, (a, si, sj, ne, ninv, c) in enumerate(_ADJS):
        contrib, idx1 = ins[2 * k], ins[2 * k + 1]
        agg = outs[k]
        ndst = _N[sj]
        nch = ne // c
        strip = st128 if c == 128 else st120
        nw0 = nch // 16
        r = nch % 16
        s0 = sid * nw0 + jnp.minimum(sid, r)
        mynw = nw0 + jnp.where(sid < r, 1, 0)
        _strip_load(idx1, strip, s0, nw0, r, sid)
        for p in range(2):
            lane0 = cid * 64 + p * 32

            def zr(rr, _):
                rows[0][rr, pl.ds(0, 16)] = zf
                rows[0][rr, pl.ds(16, 16)] = zf
                return 0

            lax.fori_loop(0, 128, zr, 0)
            bz = sid * 1880
            for j in range(14):
                pltpu.sync_copy(rows[0], spm.at[pl.ds(bz + j * 128, 128)])
            pltpu.sync_copy(rows[0].at[pl.ds(0, 88)],
                            spm.at[pl.ds(bz + 1792, 88)])
            plsc.subcore_barrier()
            ngr = mynw // 3
            rem = mynw - ngr * 3

            def grp(g, _, contrib=contrib, c=c, strip=strip, s0=s0,
                    lane0=lane0):
                for j in range(3):
                    i = g * 3 + j

                    @pl.when(g > 0)
                    def _(j=j, c=c, strip=strip, i=i):
                        pltpu.make_async_copy(
                            rows[j].at[pl.ds(0, c)], spm.at[strip.at[i]],
                            scs[j]).wait()

                    off = pl.multiple_of((s0 + i) * c, 8)
                    pltpu.async_copy(
                        contrib.at[pl.ds(off, c), pl.ds(lane0, 32)],
                        rows[j].at[pl.ds(0, c)], cps[j])
                for j in range(3):
                    i = g * 3 + j
                    off = pl.multiple_of((s0 + i) * c, 8)
                    pltpu.make_async_copy(
                        contrib.at[pl.ds(off, c), pl.ds(lane0, 32)],
                        rows[j].at[pl.ds(0, c)], cps[j]).wait()
                    pltpu.async_copy(rows[j].at[pl.ds(0, c)],
                                     spm.at[strip.at[i]], scs[j], add=True)
                return 0

            lax.fori_loop(0, ngr, grp, 0)
            for j in range(3):
                @pl.when(ngr > 0)
                def _(j=j, c=c, strip=strip):
                    pltpu.make_async_copy(
                        rows[j].at[pl.ds(0, c)], spm.at[strip.at[0]],
                        scs[j]).wait()

            def remb(t, _, contrib=contrib, c=c, strip=strip, s0=s0,
                     lane0=lane0, ngr=ngr):
                i = ngr * 3 + t
                off = pl.multiple_of((s0 + i) * c, 8)
                pltpu.async_copy(
                    contrib.at[pl.ds(off, c), pl.ds(lane0, 32)],
                    rows[0].at[pl.ds(0, c)], cps[0])
                pltpu.make_async_copy(
                    contrib.at[pl.ds(off, c), pl.ds(lane0, 32)],
                    rows[0].at[pl.ds(0, c)], cps[0]).wait()
                pltpu.sync_copy(rows[0].at[pl.ds(0, c)],
                                spm.at[strip.at[i]], add=True)
                return 0

            lax.fori_loop(0, rem, remb, 0)
            plsc.subcore_barrier()
            nchd = ndst // 40

            def dump(t, _, agg=agg, lane0=lane0):
                rr = pl.multiple_of((sid + t * 16) * 40, 8)
                pltpu.sync_copy(spm.at[pl.ds(rr, 40)],
                                agg.at[pl.ds(rr, 40), pl.ds(lane0, 32)])
                return 0

            lax.fori_loop(0, (nchd - sid + 15) // 16, dump, 0)
            plsc.subcore_barrier()


def _s2_call(args, out_type):
    fn = pl.kernel(
        _s2_body, out_type=out_type, mesh=_mesh(),
        compiler_params=_SC_PARAMS,
        scratch_types=[
            pltpu.VMEM((79, 128), i32), pltpu.VMEM((63, 120), i32),
            pltpu.VMEM((128, 32), f32), pltpu.VMEM((128, 32), f32),
            pltpu.VMEM((128, 32), f32),
            pltpu.VMEM_SHARED((_SPR, 32), f32),
            pltpu.SemaphoreType.DMA, pltpu.SemaphoreType.DMA,
            pltpu.SemaphoreType.DMA, pltpu.SemaphoreType.DMA,
            pltpu.SemaphoreType.DMA, pltpu.SemaphoreType.DMA,
        ])
    return fn(*args)


def _s3_body(pm, idx0, idx1, px, py, pz, out,
             pxv, pyv, pzv, acc, ivb0, ivb1, pmv, ridx, ash):
    cid = lax.axis_index('c')
    sid = lax.axis_index('s')

    @pl.when(cid == 0)
    def _():
        zf = jnp.zeros((16,), f32)
        pltpu.sync_copy(px, pxv)
        pltpu.sync_copy(py, pyv)
        pltpu.sync_copy(pz, pzv)

        def za(r, _):
            for g in range(8):
                acc[r, pl.ds(g * 16, 16)] = zf
            return 0

        lax.fori_loop(0, 256, za, 0)
        pltpu.sync_copy(acc.at[pl.ds(0, 16)], ash.at[pl.ds(sid * 16, 16)])
        plsc.subcore_barrier()

        def body(t, _):
            off = pl.multiple_of((sid + t * 16) * 128, 8)
            pltpu.sync_copy(idx0.at[pl.ds(off, 128)], ivb0)
            pltpu.sync_copy(idx1.at[pl.ds(off, 128)], ivb1)
            pltpu.sync_copy(pm.at[pl.ds(off, 128)], pmv)
            for g in range(8):
                s_ = ivb0[pl.ds(g * 16, 16)]
                r_ = ivb1[pl.ds(g * 16, 16)]
                w_ = pmv[pl.ds(g * 16, 16)]
                srow = lax.shift_right_logical(s_, 7)
                scol = s_ & 127
                row = lax.shift_right_logical(r_, 7)
                col = r_ & 127
                dx = (plsc.load_gather(pxv, [srow, scol])
                      - plsc.load_gather(pxv, [row, col])) * w_
                dy = (plsc.load_gather(pyv, [srow, scol])
                      - plsc.load_gather(pyv, [row, col])) * w_
                dz = (plsc.load_gather(pzv, [srow, scol])
                      - plsc.load_gather(pzv, [row, col])) * w_
                plsc.addupdate_scatter(acc, [row, col], dx)
                plsc.addupdate_scatter(acc, [row + 80, col], dy)
                plsc.addupdate_scatter(acc, [row + 160, col], dz)
            return 0

        lax.fori_loop(0, (1250 - sid + 15) // 16, body, 0)
        for t in range(2):
            for g in range(8):
                ridx[pl.ds(g * 16, 16)] = lax.iota(i32, 16) + (t * 128 + g * 16)
            pltpu.sync_copy(acc.at[pl.ds(t * 128, 128)], ash.at[ridx], add=True)
        plsc.subcore_barrier()

        @pl.when(sid == 0)
        def _dump():
            pltpu.sync_copy(ash, out)


def _s3_call(pm, idx0, idx1, px, py, pz):
    fn = pl.kernel(
        _s3_body, out_type=jax.ShapeDtypeStruct((256, H), f32), mesh=_mesh(),
        compiler_params=_SC_PARAMS,
        scratch_types=[
            pltpu.VMEM((80, H), f32), pltpu.VMEM((80, H), f32),
            pltpu.VMEM((80, H), f32), pltpu.VMEM((256, H), f32),
            pltpu.VMEM((128,), i32), pltpu.VMEM((128,), i32),
            pltpu.VMEM((128,), f32), pltpu.VMEM((128,), i32),
            pltpu.VMEM_SHARED((256, H), f32),
        ])
    return fn(pm, idx0, idx1, px, py, pz)


def _t3_body(nagg, has_vel, *refs):
    x = refs[0]
    aggs = refs[1:1 + nagg]
    u1w, u1b, u2w, u2b, g, b = refs[1 + nagg:7 + nagg]
    rest = refs[7 + nagg:]
    cat = jnp.concatenate([x[:]] + [a[:] for a in aggs], axis=1)
    h = _silu(jnp.dot(cat, u1w[:], preferred_element_type=f32) + u1b[:])
    h = jnp.dot(h, u2w[:], preferred_element_type=f32) + u2b[:]
    v = x[:] + h
    mu = jnp.mean(v, axis=1, keepdims=True)
    var = jnp.mean((v - mu) * (v - mu), axis=1, keepdims=True)
    xn = (v - mu) * lax.rsqrt(var + 1e-5) * g[:] + b[:]
    if has_vel:
        v1w, v1b, v2w, v2b, pos8, s8, vel8, xn_ref, loc_ref = rest
        xn_ref[:] = xn
        vm = jnp.dot(_silu(jnp.dot(xn, v1w[:], preferred_element_type=f32) + v1b[:]),
                     v2w[:], preferred_element_type=f32) + v2b[0, 0]
        loc_ref[:] = pos8[:] + s8[:] + vm * vel8[:]
    else:
        xn_ref, = rest
        xn_ref[:] = xn


def _t3(d, x, aggs, p, extra=None):
    n = _N[d]
    nagg = len(aggs)
    has_vel = extra is not None
    args = [x] + aggs + [
        p['up1_' + d + '_w'], p['up1_' + d + '_b'].reshape(1, H),
        p['up2_' + d + '_w'], p['up2_' + d + '_b'].reshape(1, H),
        p['ln_g_' + d].reshape(1, H), p['ln_b_' + d].reshape(1, H)]
    wspec = lambda arr: pl.BlockSpec(arr.shape, lambda i: tuple(0 for _ in arr.shape))
    in_specs = [pl.BlockSpec((_BT, H), lambda i: (i, 0))] * (1 + nagg)
    in_specs += [wspec(w) for w in args[1 + nagg:]]
    out_specs = [pl.BlockSpec((_BT, H), lambda i: (i, 0))]
    out_shape = [jax.ShapeDtypeStruct((n, H), f32)]
    if has_vel:
        pos8, s8, vel8 = extra
        args += [p['vel1_w'], p['vel1_b'].reshape(1, H),
                 p['vel2_w'], p['vel2_b'].reshape(1, 1), pos8, s8, vel8]
        in_specs += [wspec(p['vel1_w']), wspec(p['vel1_b'].reshape(1, H)),
                     wspec(p['vel2_w']), wspec(p['vel2_b'].reshape(1, 1)),
                     pl.BlockSpec((_BT, 8), lambda i: (i, 0)),
                     pl.BlockSpec((_BT, 8), lambda i: (i, 0)),
                     pl.BlockSpec((_BT, 8), lambda i: (i, 0))]
        out_specs.append(pl.BlockSpec((_BT, 8), lambda i: (i, 0)))
        out_shape.append(jax.ShapeDtypeStruct((n, 8), f32))
    return pl.pallas_call(
        functools.partial(_t3_body, nagg, has_vel),
        grid=(n // _BT,),
        in_specs=in_specs, out_specs=out_specs, out_shape=out_shape,
    )(*args)


def kernel(x_0, x_1, x_2, adj_0_0, adj_0_1, adj_1_0, adj_1_1, adj_1_2, adj_2_1,
           inv_0_0, inv_0_1, inv_1_0, inv_1_1, inv_1_2, inv_2_1,
           pos_0, vel_0, params):
    p = params
    x = {'0': x_0, '1': x_1, '2': x_2}
    adj = {'0_0': adj_0_0, '0_1': adj_0_1, '1_0': adj_1_0,
           '1_1': adj_1_1, '1_2': adj_1_2, '2_1': adj_2_1}
    inv = {'0_0': inv_0_0, '0_1': inv_0_1, '1_0': inv_1_0,
           '1_1': inv_1_1, '1_2': inv_1_2, '2_1': inv_2_1}

    groups = {'0': [], '1': [], '2': []}
    for a, si, sj, *_ in _ADJS:
        groups[si].append((a, 'i'))
        groups[sj].append((a, 'j'))
    tbl = {}
    for d in ['0', '1', '2']:
        ws = []
        for a, role in groups[d]:
            w1 = p['mp1_' + a + '_w']
            ws.append(w1[:H] if role == 'i' else w1[H:2 * H])
        outs = _t1(x[d], jnp.concatenate(ws, axis=1), len(ws))
        for (a, role), o in zip(groups[d], outs):
            tbl[(a, role)] = o

    s1_in = []
    s1_out = []
    idx2d = {a: (adj[a][0].reshape(ne // c, c), adj[a][1].reshape(ne // c, c))
             for a, si, sj, ne, ninv, c in _ADJS}
    for a, si, sj, ne, *_ in _ADJS:
        s1_in += [tbl[(a, 'i')], tbl[(a, 'j')], idx2d[a][0], idx2d[a][1]]
        s1_out += [jax.ShapeDtypeStruct((ne, H), f32)] * 2
    gath = _s1_call(s1_in, s1_out)

    contrib = {}
    pm = None
    for k, (a, si, sj, ne, ninv, c) in enumerate(_ADJS):
        args = [gath[2 * k], gath[2 * k + 1], inv[a],
                p['mp1_' + a + '_w'][2 * H:], p['mp1_' + a + '_b'].reshape(1, H),
                p['mp2_' + a + '_w'], p['mp2_' + a + '_b'].reshape(1, H),
                p['ei_' + a + '_w'], p['ei_' + a + '_b'].reshape(1, 1)]
        if a == '0_0':
            args += [p['pos1_w'], p['pos1_b'].reshape(1, H),
                     p['pos2_w'], p['pos2_b'].reshape(1, 1)]
            contrib[a], pm = _t2(a, ne, ninv, args)
            pm = pm.reshape(-1)
        else:
            contrib[a], = _t2(a, ne, ninv, args)

    s2_in = []
    s2_out = []
    for a, si, sj, ne, ninv, c in _ADJS:
        s2_in += [contrib[a], idx2d[a][1]]
        s2_out.append(jax.ShapeDtypeStruct((_N[sj], H), f32))
    aggs = _s2_call(s2_in, s2_out)
    agg = {a: aggs[k] for k, (a, *_r) in enumerate(_ADJS)}

    ppad = jnp.zeros((240,), f32)
    planes = [jnp.concatenate([pos_0[:, k], ppad]).reshape(80, H)
              for k in range(3)]
    sacc = _s3_call(pm, adj['0_0'][0], adj['0_0'][1], *planes)
    flat = sacc.reshape(-1)
    s3 = jnp.stack([flat[0:10000], flat[10240:20240], flat[20480:30480]], axis=1)

    zpad = jnp.zeros((_N['0'], 5), f32)
    pos8 = jnp.concatenate([pos_0, zpad], axis=1)
    s8 = jnp.concatenate([s3, zpad], axis=1)
    vel8 = jnp.concatenate([vel_0, zpad], axis=1)
    xn0, loc8 = _t3('0', x_0, [agg['0_0'], agg['1_0']], p, (pos8, s8, vel8))
    xn1, = _t3('1', x_1, [agg['0_1'], agg['1_1'], agg['2_1']], p)
    xn2, = _t3('2', x_2, [agg['1_2']], p)
    return xn0, xn1, xn2, loc8[:, :3]


# speedup vs baseline: 2.6295x; 1.1003x over previous
import functools

import jax
import jax.numpy as jnp
from jax import lax
from jax.experimental import pallas as pl
from jax.experimental.pallas import tpu as pltpu
from jax.experimental.pallas import tpu_sc as plsc

f32 = jnp.float32
i32 = jnp.int32

H = 128
_N = {'0': 10000, '1': 30000, '2': 10000}
_ADJS = [
    ('0_0', '0', '0', 160000, 3, 128),
    ('0_1', '0', '1', 60000, 3, 120),
    ('1_0', '1', '0', 60000, 3, 120),
    ('1_1', '1', '1', 120000, 6, 120),
    ('1_2', '1', '2', 30000, 6, 120),
    ('2_1', '2', '1', 30000, 6, 120),
]
_NW = 32
_SPR = 30080
_BT = 1000


def _silu(v):
    return v * jax.nn.sigmoid(v)


def _mesh():
    return plsc.VectorSubcoreMesh(core_axis_name='c', subcore_axis_name='s')


_SC_PARAMS = pltpu.CompilerParams(needs_layout_passes=False,
                                  use_tc_tiling_on_sc=False)


def _t1_body(nout, x_ref, w_ref, *outs):
    acc = jnp.dot(x_ref[:], w_ref[:], preferred_element_type=f32)
    for k in range(nout):
        outs[k][:] = acc[:, k * H:(k + 1) * H]


def _t1(x, wcat, nout):
    n = x.shape[0]
    return pl.pallas_call(
        functools.partial(_t1_body, nout),
        grid=(n // _BT,),
        in_specs=[pl.BlockSpec((_BT, H), lambda i: (i, 0)),
                  pl.BlockSpec(wcat.shape, lambda i: (0, 0))],
        out_specs=[pl.BlockSpec((_BT, H), lambda i: (i, 0))] * nout,
        out_shape=[jax.ShapeDtypeStruct((n, H), f32)] * nout,
    )(x, wcat)


def _strip_load(idx2d, strip, s0, nw0, r, wid):
    @pl.when(wid < r)
    def _():
        pltpu.sync_copy(idx2d.at[pl.ds(s0, nw0 + 1)], strip.at[pl.ds(0, nw0 + 1)])

    @pl.when(wid >= r)
    def _():
        pltpu.sync_copy(idx2d.at[pl.ds(s0, nw0)], strip.at[pl.ds(0, nw0)])


def _s1_body(*refs):
    ins = refs[:24]
    outs = refs[24:36]
    (st128a, st128b, st120a, st120b, ra0, ra1, ra2, rb0, rb1, rb2,
     ga0, ga1, ga2, sa0, sa1, sa2) = refs[36:]
    ras = [ra0, ra1, ra2]
    rbs = [rb0, rb1, rb2]
    gas = [ga0, ga1, ga2]
    sas = [sa0, sa1, sa2]
    wid = lax.axis_index('c') * 16 + lax.axis_index('s')
    for k, (a, si, sj, ne, ninv, c) in enumerate(_ADJS):
        tbl_i, tbl_j, idx0, idx1 = ins[4 * k:4 * k + 4]
        oi, oj = outs[2 * k:2 * k + 2]
        s0a, s1a = (st128a, st128b) if c == 128 else (st120a, st120b)
        nch = ne // c
        nw0 = nch // _NW
        r = nch % _NW
        s0 = wid * nw0 + jnp.minimum(wid, r)
        mynw = nw0 + jnp.where(wid < r, 1, 0)
        _strip_load(idx0, s0a, s0, nw0, r, wid)
        _strip_load(idx1, s1a, s0, nw0, r, wid)
        ngr = mynw // 3
        rem = mynw - ngr * 3

        def grp(g, _, tbl_i=tbl_i, tbl_j=tbl_j, oi=oi, oj=oj, c=c,
                s0a=s0a, s1a=s1a, s0=s0):
            for j in range(3):
                i = g * 3 + j

                @pl.when(g > 0)
                def _(j=j, c=c, oi=oi, oj=oj):
                    pltpu.make_async_copy(
                        ras[j].at[pl.ds(0, c)], oi.at[pl.ds(0, c)], sas[j]).wait()
                    pltpu.make_async_copy(
                        rbs[j].at[pl.ds(0, c)], oj.at[pl.ds(0, c)], sas[j]).wait()

                pltpu.async_copy(tbl_i.at[s0a.at[i]], ras[j].at[pl.ds(0, c)], gas[j])
                pltpu.async_copy(tbl_j.at[s1a.at[i]], rbs[j].at[pl.ds(0, c)], gas[j])
            for j in range(3):
                i = g * 3 + j
                pltpu.make_async_copy(
                    tbl_i.at[s0a.at[i]], ras[j].at[pl.ds(0, c)], gas[j]).wait()
                pltpu.make_async_copy(
                    tbl_j.at[s1a.at[i]], rbs[j].at[pl.ds(0, c)], gas[j]).wait()
                off = pl.multiple_of((s0 + i) * c, 8)
                pltpu.async_copy(ras[j].at[pl.ds(0, c)], oi.at[pl.ds(off, c)], sas[j])
                pltpu.async_copy(rbs[j].at[pl.ds(0, c)], oj.at[pl.ds(off, c)], sas[j])
            return 0

        lax.fori_loop(0, ngr, grp, 0)
        for j in range(3):
            @pl.when(ngr > 0)
            def _(j=j, c=c, oi=oi, oj=oj):
                pltpu.make_async_copy(
                    ras[j].at[pl.ds(0, c)], oi.at[pl.ds(0, c)], sas[j]).wait()
                pltpu.make_async_copy(
                    rbs[j].at[pl.ds(0, c)], oj.at[pl.ds(0, c)], sas[j]).wait()

        def remb(t, _, tbl_i=tbl_i, tbl_j=tbl_j, oi=oi, oj=oj, c=c,
                 s0a=s0a, s1a=s1a, s0=s0, ngr=ngr):
            i = ngr * 3 + t
            pltpu.async_copy(tbl_i.at[s0a.at[i]], ras[0].at[pl.ds(0, c)], gas[0])
            pltpu.async_copy(tbl_j.at[s1a.at[i]], rbs[0].at[pl.ds(0, c)], gas[0])
            pltpu.make_async_copy(
                tbl_i.at[s0a.at[i]], ras[0].at[pl.ds(0, c)], gas[0]).wait()
            pltpu.make_async_copy(
                tbl_j.at[s1a.at[i]], rbs[0].at[pl.ds(0, c)], gas[0]).wait()
            off = pl.multiple_of((s0 + i) * c, 8)
            pltpu.sync_copy(ras[0].at[pl.ds(0, c)], oi.at[pl.ds(off, c)])
            pltpu.sync_copy(rbs[0].at[pl.ds(0, c)], oj.at[pl.ds(off, c)])
            return 0

        lax.fori_loop(0, rem, remb, 0)


def _s1_call(args, out_type):
    fn = pl.kernel(
        _s1_body, out_type=out_type, mesh=_mesh(),
        compiler_params=_SC_PARAMS,
        scratch_types=[
            pltpu.VMEM((40, 128), i32), pltpu.VMEM((40, 128), i32),
            pltpu.VMEM((32, 120), i32), pltpu.VMEM((32, 120), i32),
            pltpu.VMEM((128, H), f32), pltpu.VMEM((128, H), f32),
            pltpu.VMEM((128, H), f32), pltpu.VMEM((128, H), f32),
            pltpu.VMEM((128, H), f32), pltpu.VMEM((128, H), f32),
            pltpu.SemaphoreType.DMA, pltpu.SemaphoreType.DMA,
            pltpu.SemaphoreType.DMA, pltpu.SemaphoreType.DMA,
            pltpu.SemaphoreType.DMA, pltpu.SemaphoreType.DMA,
        ])
    return fn(*args)


def _t2_body(has_pm, *refs):
    xi, xj, inv, w1c, b1, w2, b2, eiw, eib = refs[:9]
    s = xi[:] + xj[:] + jnp.dot(inv[:], w1c[:], preferred_element_type=f32) + b1[:]
    h1 = _silu(s)
    m = _silu(jnp.dot(h1, w2[:], preferred_element_type=f32) + b2[:])
    ew = jax.nn.sigmoid(jnp.dot(m, eiw[:], preferred_element_type=f32) + eib[0, 0])
    if has_pm:
        p1w, p1b, p2w, p2b, out, pm = refs[9:]
        t = _silu(jnp.dot(m, p1w[:], preferred_element_type=f32) + p1b[:])
        pm[:] = jnp.dot(t, p2w[:], preferred_element_type=f32) + p2b[0, 0]
    else:
        out, = refs[9:]
    out[:] = m * ew


def _t2(a, ne, ninv, args):
    has_pm = a == '0_0'
    wspec = lambda arr: pl.BlockSpec(arr.shape, lambda i: tuple(0 for _ in arr.shape))
    in_specs = [pl.BlockSpec((_BT, H), lambda i: (i, 0)),
                pl.BlockSpec((_BT, H), lambda i: (i, 0)),
                pl.BlockSpec((_BT, ninv), lambda i: (i, 0))]
    in_specs += [wspec(w) for w in args[3:]]
    out_specs = [pl.BlockSpec((_BT, H), lambda i: (i, 0))]
    out_shape = [jax.ShapeDtypeStruct((ne, H), f32)]
    if has_pm:
        out_specs.append(pl.BlockSpec((_BT, 1), lambda i: (i, 0)))
        out_shape.append(jax.ShapeDtypeStruct((ne, 1), f32))
    return pl.pallas_call(
        functools.partial(_t2_body, has_pm),
        grid=(ne // _BT,),
        in_specs=in_specs, out_specs=out_specs, out_shape=out_shape,
    )(*args)


def _s2_body(*refs):
    ins = refs[:12]
    outs = refs[12:18]
    (st128, st120, r0b, r1b, r2b, spm,
     cp0, cp1, cp2, sc0, sc1, sc2) = refs[18:]
    rows = [r0b, r1b, r2b]
    cps = [cp0, cp1, cp2]
    scs = [sc0, sc1, sc2]
    cid = lax.axis_index('c')
    sid = lax.axis_index('s')
    zf = jnp.zeros((16,), f32)

    for k, (a, si, sj, ne, ninv, c) in enumerate(_ADJS):
        contrib, idx1 = ins[2 * k], ins[2 * k + 1]
        agg = outs[k]
        ndst = _N[sj]
        nch = ne // c
        strip = st128 if c == 128 else st120
        nw0 = nch // 16
        r = nch % 16
        s0 = sid * nw0 + jnp.minimum(sid, r)
        mynw = nw0 + jnp.where(sid < r, 1, 0)
        _strip_load(idx1, strip, s0, nw0, r, sid)
        for p in range(2):
            lane0 = cid * 64 + p * 32

            def zr(rr, _):
                rows[0][rr, pl.ds(0, 16)] = zf
                rows[0][rr, pl.ds(16, 16)] = zf
                return 0

            lax.fori_loop(0, 128, zr, 0)
            bz = sid * 1880
            for j in range(14):
                pltpu.async_copy(rows[0], spm.at[pl.ds(bz + j * 128, 128)],
                                 cps[1])
            pltpu.async_copy(rows[0].at[pl.ds(0, 88)],
                             spm.at[pl.ds(bz + 1792, 88)], cps[1])
            for j in range(14):
                pltpu.make_async_copy(
                    rows[0], spm.at[pl.ds(bz, 128)], cps[1]).wait()
            pltpu.make_async_copy(
                rows[0].at[pl.ds(0, 88)], spm.at[pl.ds(bz, 88)], cps[1]).wait()
            plsc.subcore_barrier()
            ngr = mynw // 3
            rem = mynw - ngr * 3

            def grp(g, _, contrib=contrib, c=c, strip=strip, s0=s0,
                    lane0=lane0):
                for j in range(3):
                    i = g * 3 + j

                    @pl.when(g > 0)
                    def _(j=j, c=c, strip=strip, i=i):
                        pltpu.make_async_copy(
                            rows[j].at[pl.ds(0, c)], spm.at[strip.at[i]],
                            scs[j]).wait()

                    off = pl.multiple_of((s0 + i) * c, 8)
                    pltpu.async_copy(
                        contrib.at[pl.ds(off, c), pl.ds(lane0, 32)],
                        rows[j].at[pl.ds(0, c)], cps[j])
                for j in range(3):
                    i = g * 3 + j
                    off = pl.multiple_of((s0 + i) * c, 8)
                    pltpu.make_async_copy(
                        contrib.at[pl.ds(off, c), pl.ds(lane0, 32)],
                        rows[j].at[pl.ds(0, c)], cps[j]).wait()
                    pltpu.async_copy(rows[j].at[pl.ds(0, c)],
                                     spm.at[strip.at[i]], scs[j], add=True)
                return 0

            lax.fori_loop(0, ngr, grp, 0)
            for j in range(3):
                @pl.when(ngr > 0)
                def _(j=j, c=c, strip=strip):
                    pltpu.make_async_copy(
                        rows[j].at[pl.ds(0, c)], spm.at[strip.at[0]],
                        scs[j]).wait()

            def remb(t, _, contrib=contrib, c=c, strip=strip, s0=s0,
                     lane0=lane0, ngr=ngr):
                i = ngr * 3 + t
                off = pl.multiple_of((s0 + i) * c, 8)
                pltpu.async_copy(
                    contrib.at[pl.ds(off, c), pl.ds(lane0, 32)],
                    rows[0].at[pl.ds(0, c)], cps[0])
                pltpu.make_async_copy(
                    contrib.at[pl.ds(off, c), pl.ds(lane0, 32)],
                    rows[0].at[pl.ds(0, c)], cps[0]).wait()
                pltpu.sync_copy(rows[0].at[pl.ds(0, c)],
                                spm.at[strip.at[i]], add=True)
                return 0

            lax.fori_loop(0, rem, remb, 0)
            plsc.subcore_barrier()
            nchd = ndst // 40
            ndp = (nchd - sid + 15) // 16

            def dump(t, _, agg=agg, lane0=lane0):
                rr = pl.multiple_of((sid + t * 16) * 40, 8)
                pltpu.async_copy(spm.at[pl.ds(rr, 40)],
                                 agg.at[pl.ds(rr, 40), pl.ds(lane0, 32)],
                                 cps[2])
                return 0

            lax.fori_loop(0, ndp, dump, 0)

            def dumpw(t, _, agg=agg, lane0=lane0):
                pltpu.make_async_copy(
                    spm.at[pl.ds(0, 40)],
                    agg.at[pl.ds(0, 40), pl.ds(lane0, 32)], cps[2]).wait()
                return 0

            lax.fori_loop(0, ndp, dumpw, 0)
            plsc.subcore_barrier()


def _s2_call(args, out_type):
    fn = pl.kernel(
        _s2_body, out_type=out_type, mesh=_mesh(),
        compiler_params=_SC_PARAMS,
        scratch_types=[
            pltpu.VMEM((79, 128), i32), pltpu.VMEM((63, 120), i32),
            pltpu.VMEM((128, 32), f32), pltpu.VMEM((128, 32), f32),
            pltpu.VMEM((128, 32), f32),
            pltpu.VMEM_SHARED((_SPR, 32), f32),
            pltpu.SemaphoreType.DMA, pltpu.SemaphoreType.DMA,
            pltpu.SemaphoreType.DMA, pltpu.SemaphoreType.DMA,
            pltpu.SemaphoreType.DMA, pltpu.SemaphoreType.DMA,
        ])
    return fn(*args)


def _s3_body(pm, idx0, idx1, px, py, pz, out,
             pxv, pyv, pzv, acc, s0b, s1b, pmb, ridx, ash):
    cid = lax.axis_index('c')
    sid = lax.axis_index('s')

    @pl.when(cid == 0)
    def _():
        zf = jnp.zeros((16,), f32)
        pltpu.sync_copy(px, pxv)
        pltpu.sync_copy(py, pyv)
        pltpu.sync_copy(pz, pzv)
        nw0 = 1250 // 16
        r = 1250 % 16
        s0 = sid * nw0 + jnp.minimum(sid, r)
        mynw = nw0 + jnp.where(sid < r, 1, 0)
        _strip_load(idx0, s0b, s0, nw0, r, sid)
        _strip_load(idx1, s1b, s0, nw0, r, sid)
        _strip_load(pm, pmb, s0, nw0, r, sid)

        def za(rr, _):
            for g in range(8):
                acc[rr, pl.ds(g * 16, 16)] = zf
            return 0

        lax.fori_loop(0, 256, za, 0)
        pltpu.sync_copy(acc.at[pl.ds(0, 16)], ash.at[pl.ds(sid * 16, 16)])
        plsc.subcore_barrier()

        def body(i, _):
            for g in range(8):
                s_ = s0b[i, pl.ds(g * 16, 16)]
                r_ = s1b[i, pl.ds(g * 16, 16)]
                w_ = pmb[i, pl.ds(g * 16, 16)]
                srow = lax.shift_right_logical(s_, 7)
                scol = s_ & 127
                row = lax.shift_right_logical(r_, 7)
                col = r_ & 127
                dx = (plsc.load_gather(pxv, [srow, scol])
                      - plsc.load_gather(pxv, [row, col])) * w_
                dy = (plsc.load_gather(pyv, [srow, scol])
                      - plsc.load_gather(pyv, [row, col])) * w_
                dz = (plsc.load_gather(pzv, [srow, scol])
                      - plsc.load_gather(pzv, [row, col])) * w_
                plsc.addupdate_scatter(acc, [row, col], dx)
                plsc.addupdate_scatter(acc, [row + 80, col], dy)
                plsc.addupdate_scatter(acc, [row + 160, col], dz)
            return 0

        lax.fori_loop(0, mynw, body, 0)
        for t in range(2):
            for g in range(8):
                ridx[pl.ds(g * 16, 16)] = lax.iota(i32, 16) + (t * 128 + g * 16)
            pltpu.sync_copy(acc.at[pl.ds(t * 128, 128)], ash.at[ridx], add=True)
        plsc.subcore_barrier()

        @pl.when(sid == 0)
        def _dump():
            pltpu.sync_copy(ash, out)


def _s3_call(pm2d, idx0, idx1, px, py, pz):
    fn = pl.kernel(
        _s3_body, out_type=jax.ShapeDtypeStruct((256, H), f32), mesh=_mesh(),
        compiler_params=_SC_PARAMS,
        scratch_types=[
            pltpu.VMEM((80, H), f32), pltpu.VMEM((80, H), f32),
            pltpu.VMEM((80, H), f32), pltpu.VMEM((256, H), f32),
            pltpu.VMEM((79, 128), i32), pltpu.VMEM((79, 128), i32),
            pltpu.VMEM((79, 128), f32), pltpu.VMEM((128,), i32),
            pltpu.VMEM_SHARED((256, H), f32),
        ])
    return fn(pm2d, idx0, idx1, px, py, pz)


def _t3_body(nagg, has_vel, *refs):
    x = refs[0]
    aggs = refs[1:1 + nagg]
    u1w, u1b, u2w, u2b, g, b = refs[1 + nagg:7 + nagg]
    rest = refs[7 + nagg:]
    cat = jnp.concatenate([x[:]] + [a[:] for a in aggs], axis=1)
    h = _silu(jnp.dot(cat, u1w[:], preferred_element_type=f32) + u1b[:])
    h = jnp.dot(h, u2w[:], preferred_element_type=f32) + u2b[:]
    v = x[:] + h
    mu = jnp.mean(v, axis=1, keepdims=True)
    var = jnp.mean((v - mu) * (v - mu), axis=1, keepdims=True)
    xn = (v - mu) * lax.rsqrt(var + 1e-5) * g[:] + b[:]
    if has_vel:
        v1w, v1b, v2w, v2b, pos8, s8, vel8, xn_ref, loc_ref = rest
        xn_ref[:] = xn
        vm = jnp.dot(_silu(jnp.dot(xn, v1w[:], preferred_element_type=f32) + v1b[:]),
                     v2w[:], preferred_element_type=f32) + v2b[0, 0]
        loc_ref[:] = pos8[:] + s8[:] + vm * vel8[:]
    else:
        xn_ref, = rest
        xn_ref[:] = xn


def _t3(d, x, aggs, p, extra=None):
    n = _N[d]
    nagg = len(aggs)
    has_vel = extra is not None
    args = [x] + aggs + [
        p['up1_' + d + '_w'], p['up1_' + d + '_b'].reshape(1, H),
        p['up2_' + d + '_w'], p['up2_' + d + '_b'].reshape(1, H),
        p['ln_g_' + d].reshape(1, H), p['ln_b_' + d].reshape(1, H)]
    wspec = lambda arr: pl.BlockSpec(arr.shape, lambda i: tuple(0 for _ in arr.shape))
    in_specs = [pl.BlockSpec((_BT, H), lambda i: (i, 0))] * (1 + nagg)
    in_specs += [wspec(w) for w in args[1 + nagg:]]
    out_specs = [pl.BlockSpec((_BT, H), lambda i: (i, 0))]
    out_shape = [jax.ShapeDtypeStruct((n, H), f32)]
    if has_vel:
        pos8, s8, vel8 = extra
        args += [p['vel1_w'], p['vel1_b'].reshape(1, H),
                 p['vel2_w'], p['vel2_b'].reshape(1, 1), pos8, s8, vel8]
        in_specs += [wspec(p['vel1_w']), wspec(p['vel1_b'].reshape(1, H)),
                     wspec(p['vel2_w']), wspec(p['vel2_b'].reshape(1, 1)),
                     pl.BlockSpec((_BT, 8), lambda i: (i, 0)),
                     pl.BlockSpec((_BT, 8), lambda i: (i, 0)),
                     pl.BlockSpec((_BT, 8), lambda i: (i, 0))]
        out_specs.append(pl.BlockSpec((_BT, 8), lambda i: (i, 0)))
        out_shape.append(jax.ShapeDtypeStruct((n, 8), f32))
    return pl.pallas_call(
        functools.partial(_t3_body, nagg, has_vel),
        grid=(n // _BT,),
        in_specs=in_specs, out_specs=out_specs, out_shape=out_shape,
    )(*args)


def kernel(x_0, x_1, x_2, adj_0_0, adj_0_1, adj_1_0, adj_1_1, adj_1_2, adj_2_1,
           inv_0_0, inv_0_1, inv_1_0, inv_1_1, inv_1_2, inv_2_1,
           pos_0, vel_0, params):
    p = params
    x = {'0': x_0, '1': x_1, '2': x_2}
    adj = {'0_0': adj_0_0, '0_1': adj_0_1, '1_0': adj_1_0,
           '1_1': adj_1_1, '1_2': adj_1_2, '2_1': adj_2_1}
    inv = {'0_0': inv_0_0, '0_1': inv_0_1, '1_0': inv_1_0,
           '1_1': inv_1_1, '1_2': inv_1_2, '2_1': inv_2_1}

    groups = {'0': [], '1': [], '2': []}
    for a, si, sj, *_ in _ADJS:
        groups[si].append((a, 'i'))
        groups[sj].append((a, 'j'))
    tbl = {}
    for d in ['0', '1', '2']:
        ws = []
        for a, role in groups[d]:
            w1 = p['mp1_' + a + '_w']
            ws.append(w1[:H] if role == 'i' else w1[H:2 * H])
        outs = _t1(x[d], jnp.concatenate(ws, axis=1), len(ws))
        for (a, role), o in zip(groups[d], outs):
            tbl[(a, role)] = o

    s1_in = []
    s1_out = []
    idx2d = {a: (adj[a][0].reshape(ne // c, c), adj[a][1].reshape(ne // c, c))
             for a, si, sj, ne, ninv, c in _ADJS}
    for a, si, sj, ne, *_ in _ADJS:
        s1_in += [tbl[(a, 'i')], tbl[(a, 'j')], idx2d[a][0], idx2d[a][1]]
        s1_out += [jax.ShapeDtypeStruct((ne, H), f32)] * 2
    gath = _s1_call(s1_in, s1_out)

    contrib = {}
    pm = None
    for k, (a, si, sj, ne, ninv, c) in enumerate(_ADJS):
        args = [gath[2 * k], gath[2 * k + 1], inv[a],
                p['mp1_' + a + '_w'][2 * H:], p['mp1_' + a + '_b'].reshape(1, H),
                p['mp2_' + a + '_w'], p['mp2_' + a + '_b'].reshape(1, H),
                p['ei_' + a + '_w'], p['ei_' + a + '_b'].reshape(1, 1)]
        if a == '0_0':
            args += [p['pos1_w'], p['pos1_b'].reshape(1, H),
                     p['pos2_w'], p['pos2_b'].reshape(1, 1)]
            contrib[a], pm = _t2(a, ne, ninv, args)
            pm = pm.reshape(-1)
        else:
            contrib[a], = _t2(a, ne, ninv, args)

    s2_in = []
    s2_out = []
    for a, si, sj, ne, ninv, c in _ADJS:
        s2_in += [contrib[a], idx2d[a][1]]
        s2_out.append(jax.ShapeDtypeStruct((_N[sj], H), f32))
    aggs = _s2_call(s2_in, s2_out)
    agg = {a: aggs[k] for k, (a, *_r) in enumerate(_ADJS)}

    ppad = jnp.zeros((240,), f32)
    planes = [jnp.concatenate([pos_0[:, k], ppad]).reshape(80, H)
              for k in range(3)]
    sacc = _s3_call(pm.reshape(1250, 128), idx2d['0_0'][0], idx2d['0_0'][1],
                    *planes)
    flat = sacc.reshape(-1)
    s3 = jnp.stack([flat[0:10000], flat[10240:20240], flat[20480:30480]], axis=1)

    zpad = jnp.zeros((_N['0'], 5), f32)
    pos8 = jnp.concatenate([pos_0, zpad], axis=1)
    s8 = jnp.concatenate([s3, zpad], axis=1)
    vel8 = jnp.concatenate([vel_0, zpad], axis=1)
    xn0, loc8 = _t3('0', x_0, [agg['0_0'], agg['1_0']], p, (pos8, s8, vel8))
    xn1, = _t3('1', x_1, [agg['0_1'], agg['1_1'], agg['2_1']], p)
    xn2, = _t3('2', x_2, [agg['1_2']], p)
    return xn0, xn1, xn2, loc8[:, :3]
